# baseline (speedup 1.0000x reference)
import functools

import jax
import jax.numpy as jnp
from jax.experimental import pallas as pl
from jax.experimental.pallas import tpu as pltpu

V = [505, 1961, 7726]
GLOBAL_DIM = 128
LOCAL_DIM = 256


def _mm_body(x_ref, w_ref, b_ref, o_ref):
    o_ref[...] = (
        jnp.dot(x_ref[...], w_ref[...], preferred_element_type=jnp.float32)
        + b_ref[...]
    )


def _pallas_mm(x, w, b):
    n, f = x.shape
    g = w.shape[1]
    blk = 512
    n_pad = (n + blk - 1) // blk * blk
    f_pad = (f + 127) // 128 * 128
    g_pad = (g + 127) // 128 * 128
    xp = jnp.pad(x, ((0, n_pad - n), (0, f_pad - f)))
    wp = jnp.pad(w, ((0, f_pad - f), (0, g_pad - g)))
    bp = jnp.pad(b, ((0, g_pad - g),)).reshape(1, g_pad)
    out = pl.pallas_call(
        _mm_body,
        grid=(n_pad // blk,),
        in_specs=[
            pl.BlockSpec((blk, f_pad), lambda i: (i, 0)),
            pl.BlockSpec((f_pad, g_pad), lambda i: (0, 0)),
            pl.BlockSpec((1, g_pad), lambda i: (0, 0)),
        ],
        out_specs=pl.BlockSpec((blk, g_pad), lambda i: (i, 0)),
        out_shape=jax.ShapeDtypeStruct((n_pad, g_pad), jnp.float32),
    )(xp, wp, bp)
    return out[:n, :g]


def conv2d(x, w, stride):
    return jax.lax.conv_general_dilated(
        x, w, (stride, stride), "SAME", dimension_numbers=("NCHW", "OIHW", "NCHW")
    )


def gcn_conv(x, W, b, ei, num_nodes):
    src, dst = ei[0], ei[1]
    deg = jnp.zeros((num_nodes,), x.dtype).at[dst].add(1.0) + 1.0
    dinv = jax.lax.rsqrt(deg)
    norm = dinv[src] * dinv[dst]
    xt = jnp.transpose(x, (1, 0, 2))
    msgs = xt[src] * norm[:, None, None]
    agg = jax.ops.segment_sum(msgs, dst, num_segments=num_nodes)
    agg = agg + xt * (dinv * dinv)[:, None, None]
    agg = jnp.transpose(agg, (1, 0, 2))
    B, Vn, F = agg.shape
    out = _pallas_mm(agg.reshape(B * Vn, F), W, b)
    return out.reshape(B, Vn, -1)


def gbottleneck(x, p, ei, n):
    h = jax.nn.relu(gcn_conv(x, p["w_in"], p["b_in"], ei, n))
    for blk in p["blocks"]:
        h2 = jax.nn.relu(gcn_conv(h, blk["w1"], blk["b1"], ei, n))
        h2 = jax.nn.relu(gcn_conv(h2, blk["w2"], blk["b2"], ei, n))
        h = (h + h2) * 0.5
    out = gcn_conv(h, p["w_out"], p["b_out"], ei, n)
    return out, h


def gunpool(x, idx):
    mid = 0.5 * (x[:, idx[:, 0], :] + x[:, idx[:, 1], :])
    return jnp.concatenate([x, mid], axis=1)


def local_pool(x):
    B, Vn, F = x.shape
    return x.reshape(B, Vn, LOCAL_DIM, F // LOCAL_DIM).max(-1)


def bilinear(feat, u, v):
    C, H, W = feat.shape
    uf = jnp.floor(u)
    vf = jnp.floor(v)
    du = u - uf
    dv = v - vf
    u0 = jnp.clip(uf.astype(jnp.int32), 0, W - 1)
    v0 = jnp.clip(vf.astype(jnp.int32), 0, H - 1)
    u1 = jnp.clip(u0 + 1, 0, W - 1)
    v1 = jnp.clip(v0 + 1, 0, H - 1)
    f00 = feat[:, v0, u0]
    f01 = feat[:, v0, u1]
    f10 = feat[:, v1, u0]
    f11 = feat[:, v1, u1]
    out = (
        f00 * (1 - du) * (1 - dv)
        + f01 * du * (1 - dv)
        + f10 * (1 - du) * dv
        + f11 * du * dv
    )
    return out.T


def gproject(verts, local_features, data):
    C, H, W = local_features.shape[1:]
    u = jnp.clip(verts[..., 0] * data[:, 0:1] + data[:, 2:3], 0.0, 1.0) * (W - 1)
    v = jnp.clip(verts[..., 1] * data[:, 1:2] + data[:, 3:4], 0.0, 1.0) * (H - 1)
    return jax.vmap(bilinear)(local_features, u, v)


def encoder(img, p):
    h = jax.nn.relu(conv2d(img, p["c1"], 2))
    h = jax.nn.relu(conv2d(h, p["c2"], 2))
    h = jax.nn.relu(conv2d(h, p["c3"], 2))
    h = jax.nn.relu(conv2d(h, p["c4"], 2))
    local_features = h
    encode_feat = h.mean(axis=(2, 3))
    g = encode_feat @ p["wg"] + p["bg"]
    global_features = jnp.broadcast_to(g[:, None, :], (img.shape[0], V[0], GLOBAL_DIM))
    return global_features, local_features, encode_feat


def light_decoder(f, p):
    h = jax.nn.relu(f @ p["w1"] + p["b1"])
    return h @ p["w2"] + p["b2"]


def kernel(mv0, mv1, mv2, img, data, params, e0, e1, e2, up0, up1):
    B = img.shape[0]
    gf, lf, encode_feat = encoder(img, params["enc"])
    mv0b = jnp.broadcast_to(mv0[None], (B,) + mv0.shape)
    loc = local_pool(gproject(mv0b, lf, data))
    x = jnp.concatenate([mv0b, loc, gf], axis=2)
    x1, _ = gbottleneck(x, params["g0"], e0, V[0])
    loc = local_pool(gproject(mv0[None] + x1, lf, data))
    x = gunpool(jnp.concatenate([x1, loc, gf], axis=2), up0)
    x2, _ = gbottleneck(x, params["g1"], e1, V[1])
    loc = local_pool(gproject(mv1[None] + x2, lf, data))
    gf = gunpool(gf, up0)
    x = gunpool(jnp.concatenate([x2, loc, gf], axis=2), up1)
    x3h, _ = gbottleneck(x, params["g2"], e2, V[2])
    x3 = gcn_conv(x3h, params["g3"]["w"], params["g3"]["b"], e2, V[2])
    pred_pos = jnp.clip(x3 + mv2[None], 0.0, 1.0)
    loc = local_pool(gproject(mv2[None] + x3, lf, data))
    gf = gunpool(gf, up1)
    xc = jnp.concatenate([loc, gf, pred_pos], axis=2)
    c, _ = gbottleneck(xc, params["gc0"], e2, V[2])
    x3_color = gcn_conv(c, params["gc1"]["w"], params["gc1"]["b"], e2, V[2])
    pred_light = light_decoder(encode_feat, params["light"])
    return (x3, x2, x1, x3_color, pred_light)


# speedup vs baseline: 5.1916x; 5.1916x over previous
import functools

import jax
import jax.numpy as jnp
from jax import lax
from jax.experimental import pallas as pl
from jax.experimental.pallas import tpu as pltpu
import jax.experimental.pallas.tpu_sc as plsc

V = [505, 1961, 7726]
GLOBAL_DIM = 128
LOCAL_DIM = 256
_K = 64


def _rup(x, m):
    return (x + m - 1) // m * m


_M = 64
_BK = 64


@functools.cache
def _make_sc_gather(Vp, W):
    nbk = Vp // 32 // _BK
    mesh = plsc.VectorSubcoreMesh(core_axis_name="c", subcore_axis_name="s")

    def body(xs_hbm, ell_hbm, perm_hbm, degb_hbm, out_hbm,
             idx_v, pidx_v, degv, acc, buf, gsem, ssem):
        c = lax.axis_index("c")
        s = lax.axis_index("s")
        w = c * 16 + s
        pltpu.sync_copy(degb_hbm.at[w], degv)
        dvec = degv[0]
        for b in range(nbk):
            mdeg = dvec[b]
            pltpu.sync_copy(ell_hbm.at[0, w, b], idx_v)
            pltpu.async_copy(xs_hbm.at[idx_v.at[0]], acc, gsem).wait()

            @pl.loop(1, mdeg)
            def _round(r):
                pltpu.sync_copy(ell_hbm.at[r, w, b], idx_v)
                pltpu.async_copy(xs_hbm.at[idx_v.at[0]], buf, gsem).wait()

                @pl.loop(0, _BK)
                def _row(j):
                    for i in range(W // 16):
                        plsc.addupdate(acc.at[j, pl.ds(i * 16, 16)],
                                       buf[j, pl.ds(i * 16, 16)])

            pltpu.sync_copy(perm_hbm.at[w, b], pidx_v)
            pltpu.async_copy(acc, out_hbm.at[pidx_v.at[0]], ssem).wait()

    return pl.kernel(
        body,
        out_type=jax.ShapeDtypeStruct((Vp, W), jnp.float32),
        mesh=mesh,
        scratch_types=[
            pltpu.VMEM((1, _BK), jnp.int32),
            pltpu.VMEM((1, _BK), jnp.int32),
            pltpu.VMEM((1, 16), jnp.int32),
            pltpu.VMEM((_BK, W), jnp.float32),
            pltpu.VMEM((_BK, W), jnp.float32),
            pltpu.SemaphoreType.DMA,
            pltpu.SemaphoreType.DMA,
        ],
    )


def _sc_gather(xs, ell, perm, degb, Vp, W):
    return _make_sc_gather(Vp, W)(xs, ell, perm, degb)


def _mm_scale_body(x_ref, d_ref, w_ref, o_ref):
    o_ref[...] = lax.dot_general(
        x_ref[...] * d_ref[...], w_ref[...], (((1,), (0,)), ((), ())),
        preferred_element_type=jnp.float32)


def _mm_scale(x, d, w):
    n, f = x.shape
    g = w.shape[1]
    return pl.pallas_call(
        _mm_scale_body,
        grid=(n // 512,),
        in_specs=[
            pl.BlockSpec((512, f), lambda i: (i, 0)),
            pl.BlockSpec((512, 1), lambda i: (i, 0)),
            pl.BlockSpec((f, g), lambda i: (0, 0)),
        ],
        out_specs=pl.BlockSpec((512, g), lambda i: (i, 0)),
        out_shape=jax.ShapeDtypeStruct((n, g), jnp.float32),
    )(x, d, w)


def _comb_body(mode, s_ref, z_ref, d_ref, b_ref, h_ref, o_ref):
    y = d_ref[...] * (s_ref[...] + z_ref[...]) + b_ref[...]
    if mode == "relu":
        y = jnp.maximum(y, 0.0)
    elif mode == "res":
        y = (h_ref[...] + jnp.maximum(y, 0.0)) * 0.5
    o_ref[...] = y


def _combine(s, z, d, b, mode, h=None):
    n, g = z.shape
    if h is None:
        h = jnp.zeros((1, 128), jnp.float32)
        h_spec = pl.BlockSpec((1, 128), lambda i: (0, 0))
    else:
        h_spec = pl.BlockSpec((512, g), lambda i: (i, 0))
    row_spec = pl.BlockSpec((512, g), lambda i: (i, 0))
    return pl.pallas_call(
        functools.partial(_comb_body, mode),
        grid=(n // 512,),
        in_specs=[
            row_spec,
            row_spec,
            pl.BlockSpec((512, 1), lambda i: (i, 0)),
            pl.BlockSpec((1, g), lambda i: (0, 0)),
            h_spec,
        ],
        out_specs=row_spec,
        out_shape=jax.ShapeDtypeStruct((n, g), jnp.float32),
    )(s, z, d, b, h)


def _mm_body(x_ref, w_ref, b_ref, o_ref):
    o_ref[...] = lax.dot_general(
        x_ref[...], w_ref[...], (((1,), (0,)), ((), ())),
        preferred_element_type=jnp.float32) + b_ref[...]


def _pallas_mm(x, w, b):
    n, f = x.shape
    g = w.shape[1]
    blk = 256
    n_p, f_p, g_p = _rup(n, blk), _rup(f, 128), _rup(g, 128)
    xp = jnp.pad(x, ((0, n_p - n), (0, f_p - f)))
    wp = jnp.pad(w, ((0, f_p - f), (0, g_p - g)))
    bp = jnp.pad(b, ((0, g_p - g),)).reshape(1, g_p)
    out = pl.pallas_call(
        _mm_body,
        grid=(n_p // blk,),
        in_specs=[
            pl.BlockSpec((blk, f_p), lambda i: (i, 0)),
            pl.BlockSpec((f_p, g_p), lambda i: (0, 0)),
            pl.BlockSpec((1, g_p), lambda i: (0, 0)),
        ],
        out_specs=pl.BlockSpec((blk, g_p), lambda i: (i, 0)),
        out_shape=jax.ShapeDtypeStruct((n_p, g_p), jnp.float32),
    )(xp, wp, bp)
    return out[:n, :g]


class _Level:
    def __init__(self, ei, n, batch):
        self.n = n
        self.B = batch
        self.Vp = _rup(n, 32 * _BK)
        self.Np = self.Vp * batch
        src = ei[0].astype(jnp.int32)
        dst = ei[1].astype(jnp.int32)
        e = src.shape[0]
        Vp = self.Vp
        order = jnp.argsort(dst)
        ds = dst[order]
        ss = src[order]
        ar = jnp.arange(Vp, dtype=jnp.int32)
        start = jnp.searchsorted(ds, ar, side="left").astype(jnp.int32)
        end = jnp.searchsorted(ds, ar, side="right").astype(jnp.int32)
        deg = end - start
        perm = jnp.argsort(-deg).astype(jnp.int32)
        degp = deg[perm]
        rr = jnp.arange(_M, dtype=jnp.int32)[:, None]
        idxm = start[perm][None, :] + rr
        valid = rr < degp[None, :]
        zspread = (ar[None, :] * 7 + rr * 13) % (Vp - n) + n
        ell = jnp.where(valid, ss[jnp.clip(idxm, 0, e - 1)], zspread)
        nbk = Vp // 32 // _BK
        self.ell = ell.reshape(_M, 32, nbk, 1, _BK).astype(jnp.int32)
        self.perm = perm.reshape(32, nbk, 1, _BK)
        degb = degp[:: _BK].reshape(32, nbk)
        self.degb = jnp.zeros((32, 1, 16), jnp.int32).at[:, 0, :nbk].set(degb)
        dinv = lax.rsqrt(deg.astype(jnp.float32) + 1.0)
        self.dinvB = jnp.repeat(dinv, batch)[:, None]

    def aggregate(self, zs_np_g, g):
        wt = self.B * g
        assert wt % 256 == 0
        xs = zs_np_g.reshape(self.Vp, wt)
        parts = []
        for c0 in range(0, wt, 512):
            cw = min(512, wt - c0)
            parts.append(_sc_gather(xs[:, c0 : c0 + cw], self.ell, self.perm,
                                    self.degb, self.Vp, cw))
        s = parts[0] if len(parts) == 1 else jnp.concatenate(parts, axis=1)
        return s.reshape(self.Np, g)


def _gconv(x, w, b, lvl, mode, h=None):
    f = x.shape[1]
    g = w.shape[1]
    gp = _rup(g, 128)
    wp = jnp.pad(w, ((0, f - w.shape[0]), (0, gp - g)))
    zs = _mm_scale(x, lvl.dinvB, wp)
    if g >= 64:
        s = lvl.aggregate(zs[:, :g] if g < gp else zs, g)
        if g < gp:
            s = jnp.pad(s, ((0, 0), (0, gp - g)))
        bp = jnp.pad(b, ((0, gp - g),)).reshape(1, gp)
        return _combine(s, zs, lvl.dinvB, bp, mode, h)
    gpad = 256 // 4
    z4 = jnp.pad(zs[:, :g], ((0, 0), (0, gpad - g)))
    s = lvl.aggregate(z4, gpad)
    y = lvl.dinvB * (s[:, :g] + zs[:, :g]) + b[None, :]
    if mode == "relu":
        y = jnp.maximum(y, 0.0)
    elif mode == "res":
        y = (h + jnp.maximum(y, 0.0)) * 0.5
    return y


def _gbottleneck(x, p, lvl):
    h = _gconv(x, p["w_in"], p["b_in"], lvl, "relu")
    for blk in p["blocks"]:
        h2 = _gconv(h, blk["w1"], blk["b1"], lvl, "relu")
        h = _gconv(h2, blk["w2"], blk["b2"], lvl, "res", h)
    out = _gconv(h, p["w_out"], p["b_out"], lvl, "none")
    return out


def _to_rows(x_bvf, lvl, f_pad):
    b, v, f = x_bvf.shape
    xt = jnp.transpose(x_bvf, (1, 0, 2))
    xt = jnp.pad(xt, ((0, lvl.Vp - v), (0, 0), (0, f_pad - f)))
    return xt.reshape(lvl.Np, f_pad)


def _from_rows(x_np_f, lvl, g):
    return jnp.transpose(
        x_np_f.reshape(lvl.Vp, lvl.B, -1)[: lvl.n, :, :g], (1, 0, 2))


def _conv2d(x, w, stride):
    return lax.conv_general_dilated(
        x, w, (stride, stride), "SAME", dimension_numbers=("NCHW", "OIHW", "NCHW"))


def _encoder(img, p):
    h = jax.nn.relu(_conv2d(img, p["c1"], 2))
    h = jax.nn.relu(_conv2d(h, p["c2"], 2))
    h = jax.nn.relu(_conv2d(h, p["c3"], 2))
    h = jax.nn.relu(_conv2d(h, p["c4"], 2))
    encode_feat = h.mean(axis=(2, 3))
    g = _pallas_mm(encode_feat, p["wg"], p["bg"])
    return g, h, encode_feat


def _local_pool(x):
    b, vn, f = x.shape
    return x.reshape(b, vn, LOCAL_DIM, f // LOCAL_DIM).max(-1)


def _bilinear(feat, u, v):
    c, hh, ww = feat.shape
    uf = jnp.floor(u)
    vf = jnp.floor(v)
    du = u - uf
    dv = v - vf
    u0 = jnp.clip(uf.astype(jnp.int32), 0, ww - 1)
    v0 = jnp.clip(vf.astype(jnp.int32), 0, hh - 1)
    u1 = jnp.clip(u0 + 1, 0, ww - 1)
    v1 = jnp.clip(v0 + 1, 0, hh - 1)
    f00 = feat[:, v0, u0]
    f01 = feat[:, v0, u1]
    f10 = feat[:, v1, u0]
    f11 = feat[:, v1, u1]
    out = (f00 * (1 - du) * (1 - dv) + f01 * du * (1 - dv)
           + f10 * (1 - du) * dv + f11 * du * dv)
    return out.T


def _gproject(verts, local_features, data):
    c, hh, ww = local_features.shape[1:]
    u = jnp.clip(verts[..., 0] * data[:, 0:1] + data[:, 2:3], 0.0, 1.0) * (ww - 1)
    v = jnp.clip(verts[..., 1] * data[:, 1:2] + data[:, 3:4], 0.0, 1.0) * (hh - 1)
    return jax.vmap(_bilinear)(local_features, u, v)


def _gunpool(x, idx):
    mid = 0.5 * (x[:, idx[:, 0], :] + x[:, idx[:, 1], :])
    return jnp.concatenate([x, mid], axis=1)


def kernel(mv0, mv1, mv2, img, data, params, e0, e1, e2, up0, up1):
    B = img.shape[0]
    lvl0 = _Level(e0, V[0], B)
    lvl1 = _Level(e1, V[1], B)
    lvl2 = _Level(e2, V[2], B)

    g, lf, encode_feat = _encoder(img, params["enc"])
    gf0 = jnp.broadcast_to(g[:, None, :], (B, V[0], GLOBAL_DIM))
    mv0b = jnp.broadcast_to(mv0[None], (B,) + mv0.shape)

    loc = _local_pool(_gproject(mv0b, lf, data))
    x = jnp.concatenate([mv0b, loc, gf0], axis=2)
    x1r = _gbottleneck(_to_rows(x, lvl0, 512), params["g0"], lvl0)
    x1 = _from_rows(x1r, lvl0, 3)

    loc = _local_pool(_gproject(mv0[None] + x1, lf, data))
    x = _gunpool(jnp.concatenate([x1, loc, gf0], axis=2), up0)
    x2r = _gbottleneck(_to_rows(x, lvl1, 512), params["g1"], lvl1)
    x2 = _from_rows(x2r, lvl1, 3)

    gf1 = jnp.broadcast_to(g[:, None, :], (B, V[1], GLOBAL_DIM))
    loc = _local_pool(_gproject(mv1[None] + x2, lf, data))
    x = _gunpool(jnp.concatenate([x2, loc, gf1], axis=2), up1)
    x3hr = _gbottleneck(_to_rows(x, lvl2, 512), params["g2"], lvl2)
    x3r = _gconv(x3hr, params["g3"]["w"], params["g3"]["b"], lvl2, "none")
    x3 = _from_rows(x3r, lvl2, 3)

    pred_pos = jnp.clip(x3 + mv2[None], 0.0, 1.0)
    gf2 = jnp.broadcast_to(g[:, None, :], (B, V[2], GLOBAL_DIM))
    loc = _local_pool(_gproject(mv2[None] + x3, lf, data))
    xc = jnp.concatenate([loc, gf2, pred_pos], axis=2)
    cr = _gbottleneck(_to_rows(xc, lvl2, 512), params["gc0"], lvl2)
    xcr = _gconv(cr, params["gc1"]["w"], params["gc1"]["b"], lvl2, "none")
    x3_color = _from_rows(xcr, lvl2, 3)

    hli = jax.nn.relu(_pallas_mm(encode_feat, params["light"]["w1"],
                                 params["light"]["b1"]))
    pred_light = _pallas_mm(hli, params["light"]["w2"], params["light"]["b2"])
    return (x3, x2, x1, x3_color, pred_light)


# speedup vs baseline: 5.4074x; 1.0416x over previous
import functools

import jax
import jax.numpy as jnp
from jax import lax
from jax.experimental import pallas as pl
from jax.experimental.pallas import tpu as pltpu
import jax.experimental.pallas.tpu_sc as plsc

V = [505, 1961, 7726]
GLOBAL_DIM = 128
LOCAL_DIM = 256
_K = 64


def _rup(x, m):
    return (x + m - 1) // m * m


_M = 64
_BK = 64


@functools.cache
def _make_sc_gather(Vp, W):
    nbk = Vp // 32 // _BK
    mesh = plsc.VectorSubcoreMesh(core_axis_name="c", subcore_axis_name="s")

    def body(xs_hbm, ell_hbm, perm_hbm, degb_hbm, out_hbm,
             idx_v, idx2_v, pidx_v, degv, acc, bufa, bufb, ga, gb, ssem):
        c = lax.axis_index("c")
        s = lax.axis_index("s")
        w = c * 16 + s
        pltpu.sync_copy(degb_hbm.at[w], degv)
        dvec = degv[0]

        def adds(buf):
            @pl.loop(0, _BK)
            def _row(j):
                for i in range(W // 16):
                    plsc.addupdate(acc.at[j, pl.ds(i * 16, 16)],
                                   buf[j, pl.ds(i * 16, 16)])

        for b in range(nbk):
            mdeg = dvec[b]
            pltpu.sync_copy(ell_hbm.at[0, w, b], idx_v)
            pltpu.async_copy(xs_hbm.at[idx_v.at[0]], acc, ga).wait()

            @pl.loop(1, mdeg, step=2)
            def _round(r):
                r2 = jnp.minimum(r + 1, _M - 1)
                pltpu.sync_copy(ell_hbm.at[r, w, b], idx_v)
                da = pltpu.async_copy(xs_hbm.at[idx_v.at[0]], bufa, ga)
                pltpu.sync_copy(ell_hbm.at[r2, w, b], idx2_v)
                db = pltpu.async_copy(xs_hbm.at[idx2_v.at[0]], bufb, gb)
                da.wait()
                adds(bufa)
                db.wait()

                @pl.when(r + 1 < mdeg)
                def _second():
                    adds(bufb)

            pltpu.sync_copy(perm_hbm.at[w, b], pidx_v)
            pltpu.async_copy(acc, out_hbm.at[pidx_v.at[0]], ssem).wait()

    return pl.kernel(
        body,
        out_type=jax.ShapeDtypeStruct((Vp, W), jnp.float32),
        mesh=mesh,
        scratch_types=[
            pltpu.VMEM((1, _BK), jnp.int32),
            pltpu.VMEM((1, _BK), jnp.int32),
            pltpu.VMEM((1, _BK), jnp.int32),
            pltpu.VMEM((1, 16), jnp.int32),
            pltpu.VMEM((_BK, W), jnp.float32),
            pltpu.VMEM((_BK, W), jnp.float32),
            pltpu.VMEM((_BK, W), jnp.float32),
            pltpu.SemaphoreType.DMA,
            pltpu.SemaphoreType.DMA,
            pltpu.SemaphoreType.DMA,
        ],
    )


def _sc_gather(xs, ell, perm, degb, Vp, W):
    return _make_sc_gather(Vp, W)(xs, ell, perm, degb)


def _mm_scale_body(x_ref, d_ref, w_ref, o_ref):
    o_ref[...] = lax.dot_general(
        x_ref[...] * d_ref[...], w_ref[...], (((1,), (0,)), ((), ())),
        preferred_element_type=jnp.float32)


def _mm_scale(x, d, w):
    n, f = x.shape
    g = w.shape[1]
    return pl.pallas_call(
        _mm_scale_body,
        grid=(n // 512,),
        in_specs=[
            pl.BlockSpec((512, f), lambda i: (i, 0)),
            pl.BlockSpec((512, 1), lambda i: (i, 0)),
            pl.BlockSpec((f, g), lambda i: (0, 0)),
        ],
        out_specs=pl.BlockSpec((512, g), lambda i: (i, 0)),
        out_shape=jax.ShapeDtypeStruct((n, g), jnp.float32),
    )(x, d, w)


def _comb_body(mode, s_ref, z_ref, d_ref, b_ref, h_ref, o_ref):
    y = d_ref[...] * (s_ref[...] + z_ref[...]) + b_ref[...]
    if mode == "relu":
        y = jnp.maximum(y, 0.0)
    elif mode == "res":
        y = (h_ref[...] + jnp.maximum(y, 0.0)) * 0.5
    o_ref[...] = y


def _combine(s, z, d, b, mode, h=None):
    n, g = z.shape
    if h is None:
        h = jnp.zeros((1, 128), jnp.float32)
        h_spec = pl.BlockSpec((1, 128), lambda i: (0, 0))
    else:
        h_spec = pl.BlockSpec((512, g), lambda i: (i, 0))
    row_spec = pl.BlockSpec((512, g), lambda i: (i, 0))
    return pl.pallas_call(
        functools.partial(_comb_body, mode),
        grid=(n // 512,),
        in_specs=[
            row_spec,
            row_spec,
            pl.BlockSpec((512, 1), lambda i: (i, 0)),
            pl.BlockSpec((1, g), lambda i: (0, 0)),
            h_spec,
        ],
        out_specs=row_spec,
        out_shape=jax.ShapeDtypeStruct((n, g), jnp.float32),
    )(s, z, d, b, h)


def _mm_body(x_ref, w_ref, b_ref, o_ref):
    o_ref[...] = lax.dot_general(
        x_ref[...], w_ref[...], (((1,), (0,)), ((), ())),
        preferred_element_type=jnp.float32) + b_ref[...]


def _pallas_mm(x, w, b):
    n, f = x.shape
    g = w.shape[1]
    blk = 256
    n_p, f_p, g_p = _rup(n, blk), _rup(f, 128), _rup(g, 128)
    xp = jnp.pad(x, ((0, n_p - n), (0, f_p - f)))
    wp = jnp.pad(w, ((0, f_p - f), (0, g_p - g)))
    bp = jnp.pad(b, ((0, g_p - g),)).reshape(1, g_p)
    out = pl.pallas_call(
        _mm_body,
        grid=(n_p // blk,),
        in_specs=[
            pl.BlockSpec((blk, f_p), lambda i: (i, 0)),
            pl.BlockSpec((f_p, g_p), lambda i: (0, 0)),
            pl.BlockSpec((1, g_p), lambda i: (0, 0)),
        ],
        out_specs=pl.BlockSpec((blk, g_p), lambda i: (i, 0)),
        out_shape=jax.ShapeDtypeStruct((n_p, g_p), jnp.float32),
    )(xp, wp, bp)
    return out[:n, :g]


class _Level:
    def __init__(self, ei, n, batch):
        self.n = n
        self.B = batch
        self.Vp = _rup(n, 32 * _BK)
        self.Np = self.Vp * batch
        src = ei[0].astype(jnp.int32)
        dst = ei[1].astype(jnp.int32)
        e = src.shape[0]
        Vp = self.Vp
        order = jnp.argsort(dst)
        ds = dst[order]
        ss = src[order]
        ar = jnp.arange(Vp, dtype=jnp.int32)
        start = jnp.searchsorted(ds, ar, side="left").astype(jnp.int32)
        end = jnp.searchsorted(ds, ar, side="right").astype(jnp.int32)
        deg = end - start
        perm = jnp.argsort(-deg).astype(jnp.int32)
        degp = deg[perm]
        rr = jnp.arange(_M, dtype=jnp.int32)[:, None]
        idxm = start[perm][None, :] + rr
        valid = rr < degp[None, :]
        zspread = (ar[None, :] * 7 + rr * 13) % (Vp - n) + n
        ell = jnp.where(valid, ss[jnp.clip(idxm, 0, e - 1)], zspread)
        nbk = Vp // 32 // _BK
        self.ell = ell.reshape(_M, 32, nbk, 1, _BK).astype(jnp.int32)
        self.perm = perm.reshape(32, nbk, 1, _BK)
        degb = degp[:: _BK].reshape(32, nbk)
        self.degb = jnp.zeros((32, 1, 16), jnp.int32).at[:, 0, :nbk].set(degb)
        dinv = lax.rsqrt(deg.astype(jnp.float32) + 1.0)
        self.dinvB = jnp.repeat(dinv, batch)[:, None]

    def aggregate(self, zs_np_g, g):
        wt = self.B * g
        assert wt % 256 == 0
        xs = zs_np_g.reshape(self.Vp, wt)
        parts = []
        for c0 in range(0, wt, 512):
            cw = min(512, wt - c0)
            parts.append(_sc_gather(xs[:, c0 : c0 + cw], self.ell, self.perm,
                                    self.degb, self.Vp, cw))
        s = parts[0] if len(parts) == 1 else jnp.concatenate(parts, axis=1)
        return s.reshape(self.Np, g)


def _gconv(x, w, b, lvl, mode, h=None):
    f = x.shape[1]
    g = w.shape[1]
    gp = _rup(g, 128)
    wp = jnp.pad(w, ((0, f - w.shape[0]), (0, gp - g)))
    zs = _mm_scale(x, lvl.dinvB, wp)
    if g >= 64:
        s = lvl.aggregate(zs[:, :g] if g < gp else zs, g)
        if g < gp:
            s = jnp.pad(s, ((0, 0), (0, gp - g)))
        bp = jnp.pad(b, ((0, gp - g),)).reshape(1, gp)
        return _combine(s, zs, lvl.dinvB, bp, mode, h)
    gpad = 256 // 4
    z4 = jnp.pad(zs[:, :g], ((0, 0), (0, gpad - g)))
    s = lvl.aggregate(z4, gpad)
    y = lvl.dinvB * (s[:, :g] + zs[:, :g]) + b[None, :]
    if mode == "relu":
        y = jnp.maximum(y, 0.0)
    elif mode == "res":
        y = (h + jnp.maximum(y, 0.0)) * 0.5
    return y


def _gbottleneck(x, p, lvl):
    h = _gconv(x, p["w_in"], p["b_in"], lvl, "relu")
    for blk in p["blocks"]:
        h2 = _gconv(h, blk["w1"], blk["b1"], lvl, "relu")
        h = _gconv(h2, blk["w2"], blk["b2"], lvl, "res", h)
    out = _gconv(h, p["w_out"], p["b_out"], lvl, "none")
    return out


def _to_rows(x_bvf, lvl, f_pad):
    b, v, f = x_bvf.shape
    xt = jnp.transpose(x_bvf, (1, 0, 2))
    xt = jnp.pad(xt, ((0, lvl.Vp - v), (0, 0), (0, f_pad - f)))
    return xt.reshape(lvl.Np, f_pad)


def _from_rows(x_np_f, lvl, g):
    return jnp.transpose(
        x_np_f.reshape(lvl.Vp, lvl.B, -1)[: lvl.n, :, :g], (1, 0, 2))


def _conv2d(x, w, stride):
    return lax.conv_general_dilated(
        x, w, (stride, stride), "SAME", dimension_numbers=("NCHW", "OIHW", "NCHW"))


def _encoder(img, p):
    h = jax.nn.relu(_conv2d(img, p["c1"], 2))
    h = jax.nn.relu(_conv2d(h, p["c2"], 2))
    h = jax.nn.relu(_conv2d(h, p["c3"], 2))
    h = jax.nn.relu(_conv2d(h, p["c4"], 2))
    encode_feat = h.mean(axis=(2, 3))
    g = _pallas_mm(encode_feat, p["wg"], p["bg"])
    return g, h, encode_feat


def _local_pool(x):
    b, vn, f = x.shape
    return x.reshape(b, vn, LOCAL_DIM, f // LOCAL_DIM).max(-1)


def _bilinear(feat, u, v):
    c, hh, ww = feat.shape
    uf = jnp.floor(u)
    vf = jnp.floor(v)
    du = u - uf
    dv = v - vf
    u0 = jnp.clip(uf.astype(jnp.int32), 0, ww - 1)
    v0 = jnp.clip(vf.astype(jnp.int32), 0, hh - 1)
    u1 = jnp.clip(u0 + 1, 0, ww - 1)
    v1 = jnp.clip(v0 + 1, 0, hh - 1)
    f00 = feat[:, v0, u0]
    f01 = feat[:, v0, u1]
    f10 = feat[:, v1, u0]
    f11 = feat[:, v1, u1]
    out = (f00 * (1 - du) * (1 - dv) + f01 * du * (1 - dv)
           + f10 * (1 - du) * dv + f11 * du * dv)
    return out.T


def _gproject(verts, local_features, data):
    c, hh, ww = local_features.shape[1:]
    u = jnp.clip(verts[..., 0] * data[:, 0:1] + data[:, 2:3], 0.0, 1.0) * (ww - 1)
    v = jnp.clip(verts[..., 1] * data[:, 1:2] + data[:, 3:4], 0.0, 1.0) * (hh - 1)
    return jax.vmap(_bilinear)(local_features, u, v)


def _gunpool(x, idx):
    mid = 0.5 * (x[:, idx[:, 0], :] + x[:, idx[:, 1], :])
    return jnp.concatenate([x, mid], axis=1)


def kernel(mv0, mv1, mv2, img, data, params, e0, e1, e2, up0, up1):
    B = img.shape[0]
    lvl0 = _Level(e0, V[0], B)
    lvl1 = _Level(e1, V[1], B)
    lvl2 = _Level(e2, V[2], B)

    g, lf, encode_feat = _encoder(img, params["enc"])
    gf0 = jnp.broadcast_to(g[:, None, :], (B, V[0], GLOBAL_DIM))
    mv0b = jnp.broadcast_to(mv0[None], (B,) + mv0.shape)

    loc = _local_pool(_gproject(mv0b, lf, data))
    x = jnp.concatenate([mv0b, loc, gf0], axis=2)
    x1r = _gbottleneck(_to_rows(x, lvl0, 512), params["g0"], lvl0)
    x1 = _from_rows(x1r, lvl0, 3)

    loc = _local_pool(_gproject(mv0[None] + x1, lf, data))
    x = _gunpool(jnp.concatenate([x1, loc, gf0], axis=2), up0)
    x2r = _gbottleneck(_to_rows(x, lvl1, 512), params["g1"], lvl1)
    x2 = _from_rows(x2r, lvl1, 3)

    gf1 = jnp.broadcast_to(g[:, None, :], (B, V[1], GLOBAL_DIM))
    loc = _local_pool(_gproject(mv1[None] + x2, lf, data))
    x = _gunpool(jnp.concatenate([x2, loc, gf1], axis=2), up1)
    x3hr = _gbottleneck(_to_rows(x, lvl2, 512), params["g2"], lvl2)
    x3r = _gconv(x3hr, params["g3"]["w"], params["g3"]["b"], lvl2, "none")
    x3 = _from_rows(x3r, lvl2, 3)

    pred_pos = jnp.clip(x3 + mv2[None], 0.0, 1.0)
    gf2 = jnp.broadcast_to(g[:, None, :], (B, V[2], GLOBAL_DIM))
    loc = _local_pool(_gproject(mv2[None] + x3, lf, data))
    xc = jnp.concatenate([loc, gf2, pred_pos], axis=2)
    cr = _gbottleneck(_to_rows(xc, lvl2, 512), params["gc0"], lvl2)
    xcr = _gconv(cr, params["gc1"]["w"], params["gc1"]["b"], lvl2, "none")
    x3_color = _from_rows(xcr, lvl2, 3)

    hli = jax.nn.relu(_pallas_mm(encode_feat, params["light"]["w1"],
                                 params["light"]["b1"]))
    pred_light = _pallas_mm(hli, params["light"]["w2"], params["light"]["b2"])
    return (x3, x2, x1, x3_color, pred_light)
